# baseline (device time: 35484 ns/iter reference)
import jax
import jax.numpy as jnp
from jax import lax
from jax.experimental import pallas as pl
from jax.experimental.pallas import tpu as pltpu

N_DEV = 4
B = 2
SQ = 128
DH = 64
H_LOC = 4
D_LOC = H_LOC * DH
D_MODEL = 512


def kernel(x, Wq, K_ext, V_ext, Wo):
    K_t = jnp.transpose(K_ext, (0, 2, 1, 3))
    V_t = jnp.transpose(V_ext, (0, 2, 1, 3))

    def body(x_ref, wq_ref, k_ref, v_ref, wo_ref, out_ref,
             comm_ref, send_sems, recv_sems):
        my_pos = lax.axis_index("i")
        left = lax.rem(my_pos + N_DEV - 1, N_DEV)
        right = lax.rem(my_pos + 1, N_DEV)

        barrier_sem = pltpu.get_barrier_semaphore()
        for nbr in (left, right):
            pl.semaphore_signal(
                barrier_sem, inc=1,
                device_id=(nbr,), device_id_type=pl.DeviceIdType.MESH,
            )
        pl.semaphore_wait(barrier_sem, 2)

        col0 = my_pos * D_LOC
        for b in range(B):
            xb = x_ref[b]
            q_all = jnp.dot(xb, wq_ref[:, pl.ds(col0, D_LOC)],
                            preferred_element_type=jnp.float32)
            ctxs = []
            for j in range(H_LOC):
                q = q_all[:, j * DH:(j + 1) * DH]
                k = k_ref[b, j]
                v = v_ref[b, j]
                s = lax.dot_general(
                    q, k, (((1,), (1,)), ((), ())),
                    preferred_element_type=jnp.float32) * 0.125
                m = jnp.max(s, axis=-1, keepdims=True)
                e = jnp.exp(s - m)
                w = e / jnp.sum(e, axis=-1, keepdims=True)
                ctxs.append(jnp.dot(w, v, preferred_element_type=jnp.float32))
            ctx_all = jnp.concatenate(ctxs, axis=1)
            part = jnp.dot(ctx_all, wo_ref[pl.ds(col0, D_LOC), :],
                           preferred_element_type=jnp.float32)
            out_ref[b] = part
            comm_ref[0, b] = part

        for h in range(N_DEV - 1):
            rdma = pltpu.make_async_remote_copy(
                src_ref=comm_ref.at[h],
                dst_ref=comm_ref.at[h + 1],
                send_sem=send_sems.at[h],
                recv_sem=recv_sems.at[h],
                device_id=(right,),
                device_id_type=pl.DeviceIdType.MESH,
            )
            rdma.start()
            rdma.wait()
            out_ref[:] = out_ref[:] + comm_ref[h + 1]

    return pl.pallas_call(
        body,
        out_shape=jax.ShapeDtypeStruct((B, SQ, D_MODEL), jnp.float32),
        in_specs=[pl.BlockSpec(memory_space=pltpu.VMEM)] * 5,
        out_specs=pl.BlockSpec(memory_space=pltpu.VMEM),
        scratch_shapes=[
            pltpu.VMEM((N_DEV, B, SQ, D_MODEL), jnp.float32),
            pltpu.SemaphoreType.DMA((N_DEV - 1,)),
            pltpu.SemaphoreType.DMA((N_DEV - 1,)),
        ],
        compiler_params=pltpu.CompilerParams(collective_id=0),
    )(x, Wq, K_t, V_t, Wo)


# device time: 20587 ns/iter; 1.7236x vs baseline; 1.7236x over previous
import jax
import jax.numpy as jnp
from jax import lax
from jax.experimental import pallas as pl
from jax.experimental.pallas import tpu as pltpu

N_DEV = 4
B = 2
SQ = 128
DH = 64
H_LOC = 4
D_LOC = H_LOC * DH
D_MODEL = 512


def kernel(x, Wq, K_ext, V_ext, Wo):
    K_r = jnp.reshape(K_ext, (B, SQ, D_LOC))
    V_r = jnp.reshape(V_ext, (B, SQ, D_LOC))

    def body(x_ref, wq_ref, k_ref, v_ref, wo_ref, out_ref,
             comm_ref, send_sems, recv_sems):
        my_pos = lax.axis_index("i")
        left = lax.rem(my_pos + N_DEV - 1, N_DEV)
        right = lax.rem(my_pos + 1, N_DEV)
        opp = lax.rem(my_pos + 2, N_DEV)

        barrier_sem = pltpu.get_barrier_semaphore()
        for nbr in (left, right):
            pl.semaphore_signal(
                barrier_sem, inc=1,
                device_id=(nbr,), device_id_type=pl.DeviceIdType.MESH,
            )
        pl.semaphore_wait(barrier_sem, 2)

        col0 = my_pos * D_LOC
        ctx = []
        for b in range(B):
            xb = x_ref[b]
            q_all = jnp.dot(xb, wq_ref[:, pl.ds(col0, D_LOC)],
                            preferred_element_type=jnp.float32)
            ctxs = []
            for j in range(H_LOC):
                q = q_all[:, j * DH:(j + 1) * DH]
                k = k_ref[b, :, j * DH:(j + 1) * DH]
                v = v_ref[b, :, j * DH:(j + 1) * DH]
                s = lax.dot_general(
                    q, k, (((1,), (1,)), ((), ())),
                    preferred_element_type=jnp.float32) * 0.125
                m = jnp.max(s, axis=-1, keepdims=True)
                e = jnp.exp(s - m)
                w = e / jnp.sum(e, axis=-1, keepdims=True)
                ctxs.append(jnp.dot(w, v, preferred_element_type=jnp.float32))
            cb = jnp.concatenate(ctxs, axis=1)
            ctx.append(cb)
            comm_ref[0, b] = cb

        s0 = pltpu.make_async_remote_copy(
            src_ref=comm_ref.at[0], dst_ref=comm_ref.at[1],
            send_sem=send_sems.at[0], recv_sem=recv_sems.at[0],
            device_id=(right,), device_id_type=pl.DeviceIdType.MESH,
        )
        s1 = pltpu.make_async_remote_copy(
            src_ref=comm_ref.at[0], dst_ref=comm_ref.at[2],
            send_sem=send_sems.at[1], recv_sem=recv_sems.at[1],
            device_id=(left,), device_id_type=pl.DeviceIdType.MESH,
        )
        s0.start()
        s1.start()

        acc = [jnp.dot(ctx[b], wo_ref[pl.ds(col0, D_LOC), :],
                       preferred_element_type=jnp.float32)
               for b in range(B)]

        s3 = pltpu.make_async_remote_copy(
            src_ref=comm_ref.at[1, 1], dst_ref=comm_ref.at[3, 1],
            send_sem=send_sems.at[3], recv_sem=recv_sems.at[3],
            device_id=(right,), device_id_type=pl.DeviceIdType.MESH,
        )
        s2 = pltpu.make_async_remote_copy(
            src_ref=comm_ref.at[2, 0], dst_ref=comm_ref.at[3, 0],
            send_sem=send_sems.at[2], recv_sem=recv_sems.at[2],
            device_id=(left,), device_id_type=pl.DeviceIdType.MESH,
        )
        s0.wait_recv()
        s3.start()
        s1.wait_recv()
        s2.start()

        for b in range(B):
            acc[b] = acc[b] + jnp.dot(
                comm_ref[1, b], wo_ref[pl.ds(left * D_LOC, D_LOC), :],
                preferred_element_type=jnp.float32)
            acc[b] = acc[b] + jnp.dot(
                comm_ref[2, b], wo_ref[pl.ds(right * D_LOC, D_LOC), :],
                preferred_element_type=jnp.float32)

        s2.wait_recv()
        s3.wait_recv()
        for b in range(B):
            acc[b] = acc[b] + jnp.dot(
                comm_ref[3, b], wo_ref[pl.ds(opp * D_LOC, D_LOC), :],
                preferred_element_type=jnp.float32)
            out_ref[b] = acc[b]

        s0.wait_send()
        s1.wait_send()
        s2.wait_send()
        s3.wait_send()

    return pl.pallas_call(
        body,
        out_shape=jax.ShapeDtypeStruct((B, SQ, D_MODEL), jnp.float32),
        in_specs=[pl.BlockSpec(memory_space=pltpu.VMEM)] * 5,
        out_specs=pl.BlockSpec(memory_space=pltpu.VMEM),
        scratch_shapes=[
            pltpu.VMEM((4, B, SQ, D_LOC), jnp.float32),
            pltpu.SemaphoreType.DMA((4,)),
            pltpu.SemaphoreType.DMA((4,)),
        ],
        compiler_params=pltpu.CompilerParams(collective_id=0),
    )(x, Wq, K_r, V_r, Wo)


# device time: 18615 ns/iter; 1.9062x vs baseline; 1.1059x over previous
import jax
import jax.numpy as jnp
from jax import lax
from jax.experimental import pallas as pl
from jax.experimental.pallas import tpu as pltpu

N_DEV = 4
B = 2
SQ = 128
DH = 64
H_LOC = 4
D_LOC = H_LOC * DH
D_MODEL = 512


def kernel(x, Wq, K_ext, V_ext, Wo):
    K_r = jnp.reshape(K_ext, (B, SQ, D_LOC))
    V_r = jnp.reshape(V_ext, (B, SQ, D_LOC))

    def body(x_ref, wq_ref, k_ref, v_ref, wo_ref, out_ref,
             comm_ref, send_sems, recv_sems):
        my_pos = lax.axis_index("i")
        left = lax.rem(my_pos + N_DEV - 1, N_DEV)
        right = lax.rem(my_pos + 1, N_DEV)
        opp = lax.rem(my_pos + 2, N_DEV)

        barrier_sem = pltpu.get_barrier_semaphore()
        for nbr in (left, right):
            pl.semaphore_signal(
                barrier_sem, inc=1,
                device_id=(nbr,), device_id_type=pl.DeviceIdType.MESH,
            )
        pl.semaphore_wait(barrier_sem, 2)

        col0 = my_pos * D_LOC
        ctx = []
        for b in range(B):
            xb = x_ref[b].astype(jnp.bfloat16)
            wq_b = wq_ref[:, pl.ds(col0, D_LOC)].astype(jnp.bfloat16)
            q_all = jnp.dot(xb, wq_b,
                            preferred_element_type=jnp.float32)
            q_all = q_all.astype(jnp.bfloat16)
            kb = k_ref[b].astype(jnp.bfloat16)
            vb = v_ref[b].astype(jnp.bfloat16)
            ctxs = []
            for j in range(H_LOC):
                q = q_all[:, j * DH:(j + 1) * DH]
                k = kb[:, j * DH:(j + 1) * DH]
                v = vb[:, j * DH:(j + 1) * DH]
                s = lax.dot_general(
                    q, k, (((1,), (1,)), ((), ())),
                    preferred_element_type=jnp.float32) * 0.125
                m = jnp.max(s, axis=-1, keepdims=True)
                e = jnp.exp(s - m)
                w = (e / jnp.sum(e, axis=-1, keepdims=True)).astype(jnp.bfloat16)
                ctxs.append(jnp.dot(w, v, preferred_element_type=jnp.float32))
            cb = jnp.concatenate(ctxs, axis=1).astype(jnp.bfloat16)
            ctx.append(cb)
            comm_ref[0, b] = cb

        s0 = pltpu.make_async_remote_copy(
            src_ref=comm_ref.at[0], dst_ref=comm_ref.at[1],
            send_sem=send_sems.at[0], recv_sem=recv_sems.at[0],
            device_id=(right,), device_id_type=pl.DeviceIdType.MESH,
        )
        s1 = pltpu.make_async_remote_copy(
            src_ref=comm_ref.at[0], dst_ref=comm_ref.at[2],
            send_sem=send_sems.at[1], recv_sem=recv_sems.at[1],
            device_id=(left,), device_id_type=pl.DeviceIdType.MESH,
        )
        s0.start()
        s1.start()

        wo_own = wo_ref[pl.ds(col0, D_LOC), :].astype(jnp.bfloat16)
        acc = [jnp.dot(ctx[b], wo_own,
                       preferred_element_type=jnp.float32)
               for b in range(B)]

        s3 = pltpu.make_async_remote_copy(
            src_ref=comm_ref.at[1, 1], dst_ref=comm_ref.at[3, 1],
            send_sem=send_sems.at[3], recv_sem=recv_sems.at[3],
            device_id=(right,), device_id_type=pl.DeviceIdType.MESH,
        )
        s2 = pltpu.make_async_remote_copy(
            src_ref=comm_ref.at[2, 0], dst_ref=comm_ref.at[3, 0],
            send_sem=send_sems.at[2], recv_sem=recv_sems.at[2],
            device_id=(left,), device_id_type=pl.DeviceIdType.MESH,
        )
        s0.wait_recv()
        s3.start()
        s1.wait_recv()
        s2.start()

        wo_left = wo_ref[pl.ds(left * D_LOC, D_LOC), :].astype(jnp.bfloat16)
        wo_right = wo_ref[pl.ds(right * D_LOC, D_LOC), :].astype(jnp.bfloat16)
        for b in range(B):
            acc[b] = acc[b] + jnp.dot(
                comm_ref[1, b], wo_left,
                preferred_element_type=jnp.float32)
            acc[b] = acc[b] + jnp.dot(
                comm_ref[2, b], wo_right,
                preferred_element_type=jnp.float32)

        s2.wait_recv()
        s3.wait_recv()
        wo_opp = wo_ref[pl.ds(opp * D_LOC, D_LOC), :].astype(jnp.bfloat16)
        for b in range(B):
            acc[b] = acc[b] + jnp.dot(
                comm_ref[3, b], wo_opp,
                preferred_element_type=jnp.float32)
            out_ref[b] = acc[b]

        s0.wait_send()
        s1.wait_send()
        s2.wait_send()
        s3.wait_send()

    return pl.pallas_call(
        body,
        out_shape=jax.ShapeDtypeStruct((B, SQ, D_MODEL), jnp.float32),
        in_specs=[pl.BlockSpec(memory_space=pltpu.VMEM)] * 5,
        out_specs=pl.BlockSpec(memory_space=pltpu.VMEM),
        scratch_shapes=[
            pltpu.VMEM((4, B, SQ, D_LOC), jnp.bfloat16),
            pltpu.SemaphoreType.DMA((4,)),
            pltpu.SemaphoreType.DMA((4,)),
        ],
        compiler_params=pltpu.CompilerParams(collective_id=0),
    )(x, Wq, K_r, V_r, Wo)


# device time: 18328 ns/iter; 1.9361x vs baseline; 1.0157x over previous
import jax
import jax.numpy as jnp
from jax import lax
from jax.experimental import pallas as pl
from jax.experimental.pallas import tpu as pltpu

N_DEV = 4
B = 2
SQ = 128
DH = 64
H_LOC = 4
D_LOC = H_LOC * DH
D_MODEL = 512


def kernel(x, Wq, K_ext, V_ext, Wo):
    K_t = jnp.transpose(K_ext, (0, 2, 3, 1))
    V_t = jnp.transpose(V_ext, (0, 2, 3, 1))

    def body(x_hbm, wq_hbm, k_hbm, v_hbm, wo_hbm, out_hbm,
             xv, wqv, kv, vv, wov, outv, comm_ref,
             dma_sems, send_sems, recv_sems):
        my_pos = lax.axis_index("i")
        left = lax.rem(my_pos + N_DEV - 1, N_DEV)
        right = lax.rem(my_pos + 1, N_DEV)
        opp = lax.rem(my_pos + 2, N_DEV)
        col0 = my_pos * D_LOC

        cp_x = pltpu.make_async_copy(x_hbm, xv, dma_sems.at[0])
        cp_x.start()
        cp_wq = pltpu.make_async_copy(
            wq_hbm.at[:, pl.ds(col0, D_LOC)], wqv, dma_sems.at[1])
        cp_wq.start()
        cp_k = pltpu.make_async_copy(k_hbm, kv, dma_sems.at[2])
        cp_k.start()
        cp_v = pltpu.make_async_copy(v_hbm, vv, dma_sems.at[3])
        cp_v.start()
        cp_wo = []
        for t, org in enumerate((my_pos, left, right, opp)):
            c = pltpu.make_async_copy(
                wo_hbm.at[pl.ds(org * D_LOC, D_LOC), :], wov.at[t],
                dma_sems.at[4 + t])
            c.start()
            cp_wo.append(c)

        barrier_sem = pltpu.get_barrier_semaphore()
        for nbr in (left, right):
            pl.semaphore_signal(
                barrier_sem, inc=1,
                device_id=(nbr,), device_id_type=pl.DeviceIdType.MESH,
            )
        pl.semaphore_wait(barrier_sem, 2)

        cp_x.wait()
        cp_wq.wait()
        cp_k.wait()
        cp_v.wait()
        wq_bf = wqv[:].astype(jnp.bfloat16)
        ctx = []
        for b in range(B):
            xb = xv[b].astype(jnp.bfloat16)
            q_all = jnp.dot(xb, wq_bf,
                            preferred_element_type=jnp.float32)
            q_all = q_all.astype(jnp.bfloat16)
            ctxs = []
            for j in range(H_LOC):
                q = q_all[:, j * DH:(j + 1) * DH]
                kT = kv[b, j].astype(jnp.bfloat16)
                vT = vv[b, j].astype(jnp.bfloat16)
                s = lax.dot_general(
                    q, kT, (((1,), (0,)), ((), ())),
                    preferred_element_type=jnp.float32) * 0.125
                m = jnp.max(s, axis=-1, keepdims=True)
                e = jnp.exp(s - m)
                w = (e / jnp.sum(e, axis=-1, keepdims=True)).astype(jnp.bfloat16)
                ctxs.append(lax.dot_general(
                    w, vT, (((1,), (1,)), ((), ())),
                    preferred_element_type=jnp.float32))
            cb = jnp.concatenate(ctxs, axis=1).astype(jnp.bfloat16)
            ctx.append(cb)
            comm_ref[0, b] = cb

        s0 = pltpu.make_async_remote_copy(
            src_ref=comm_ref.at[0], dst_ref=comm_ref.at[1],
            send_sem=send_sems.at[0], recv_sem=recv_sems.at[0],
            device_id=(right,), device_id_type=pl.DeviceIdType.MESH,
        )
        s1 = pltpu.make_async_remote_copy(
            src_ref=comm_ref.at[0], dst_ref=comm_ref.at[2],
            send_sem=send_sems.at[1], recv_sem=recv_sems.at[1],
            device_id=(left,), device_id_type=pl.DeviceIdType.MESH,
        )
        s0.start()
        s1.start()

        cp_wo[0].wait()
        wo_own = wov[0].astype(jnp.bfloat16)
        acc = [jnp.dot(ctx[b], wo_own, preferred_element_type=jnp.float32)
               for b in range(B)]

        s3 = pltpu.make_async_remote_copy(
            src_ref=comm_ref.at[1, 1], dst_ref=comm_ref.at[3, 1],
            send_sem=send_sems.at[3], recv_sem=recv_sems.at[3],
            device_id=(right,), device_id_type=pl.DeviceIdType.MESH,
        )
        s2 = pltpu.make_async_remote_copy(
            src_ref=comm_ref.at[2, 0], dst_ref=comm_ref.at[3, 0],
            send_sem=send_sems.at[2], recv_sem=recv_sems.at[2],
            device_id=(left,), device_id_type=pl.DeviceIdType.MESH,
        )
        s0.wait_recv()
        s3.start()
        s1.wait_recv()
        s2.start()

        cp_wo[1].wait()
        cp_wo[2].wait()
        wo_left = wov[1].astype(jnp.bfloat16)
        wo_right = wov[2].astype(jnp.bfloat16)
        for b in range(B):
            acc[b] = acc[b] + jnp.dot(
                comm_ref[1, b], wo_left, preferred_element_type=jnp.float32)
            acc[b] = acc[b] + jnp.dot(
                comm_ref[2, b], wo_right, preferred_element_type=jnp.float32)

        s2.wait_recv()
        s3.wait_recv()
        cp_wo[3].wait()
        wo_opp = wov[3].astype(jnp.bfloat16)
        for b in range(B):
            acc[b] = acc[b] + jnp.dot(
                comm_ref[3, b], wo_opp, preferred_element_type=jnp.float32)
            outv[b] = acc[b]

        cp_out = pltpu.make_async_copy(outv, out_hbm, dma_sems.at[8])
        cp_out.start()
        cp_out.wait()

        s0.wait_send()
        s1.wait_send()
        s2.wait_send()
        s3.wait_send()

    return pl.pallas_call(
        body,
        out_shape=jax.ShapeDtypeStruct((B, SQ, D_MODEL), jnp.float32),
        in_specs=[pl.BlockSpec(memory_space=pl.ANY)] * 5,
        out_specs=pl.BlockSpec(memory_space=pl.ANY),
        scratch_shapes=[
            pltpu.VMEM((B, SQ, D_MODEL), jnp.float32),
            pltpu.VMEM((D_MODEL, D_LOC), jnp.float32),
            pltpu.VMEM((B, H_LOC, DH, SQ), jnp.float32),
            pltpu.VMEM((B, H_LOC, DH, SQ), jnp.float32),
            pltpu.VMEM((4, D_LOC, D_MODEL), jnp.float32),
            pltpu.VMEM((B, SQ, D_MODEL), jnp.float32),
            pltpu.VMEM((4, B, SQ, D_LOC), jnp.bfloat16),
            pltpu.SemaphoreType.DMA((9,)),
            pltpu.SemaphoreType.DMA((4,)),
            pltpu.SemaphoreType.DMA((4,)),
        ],
        compiler_params=pltpu.CompilerParams(collective_id=0),
    )(x, Wq, K_t, V_t, Wo)


# device time: 13027 ns/iter; 2.7239x vs baseline; 1.4069x over previous
import jax
import jax.numpy as jnp
from jax import lax
from jax.experimental import pallas as pl
from jax.experimental.pallas import tpu as pltpu

N_DEV = 4
B = 2
SQ = 128
DH = 64
H_LOC = 4
D_LOC = H_LOC * DH
D_MODEL = 512


def kernel(x, Wq, K_ext, V_ext, Wo):
    K_t = jnp.transpose(K_ext, (0, 2, 3, 1))
    V_t = jnp.transpose(V_ext, (0, 2, 3, 1))

    hbm = lambda a: pltpu.with_memory_space_constraint(a, pltpu.MemorySpace.HBM)
    x, Wq, K_t, V_t, Wo = hbm(x), hbm(Wq), hbm(K_t), hbm(V_t), hbm(Wo)

    def body(x_hbm, wq_hbm, k_hbm, v_hbm, wo_hbm, out_hbm,
             xv, wqv, kv, vv, wov, outv, comm_ref,
             dma_sems, send_sems, recv_sems):
        my_pos = lax.axis_index("i")
        left = lax.rem(my_pos + N_DEV - 1, N_DEV)
        right = lax.rem(my_pos + 1, N_DEV)
        opp = lax.rem(my_pos + 2, N_DEV)
        col0 = my_pos * D_LOC

        cp_x = pltpu.make_async_copy(x_hbm, xv, dma_sems.at[0])
        cp_x.start()
        cp_wq = pltpu.make_async_copy(
            wq_hbm.at[:, pl.ds(col0, D_LOC)], wqv, dma_sems.at[1])
        cp_wq.start()
        cp_k = pltpu.make_async_copy(k_hbm, kv, dma_sems.at[2])
        cp_k.start()
        cp_v = pltpu.make_async_copy(v_hbm, vv, dma_sems.at[3])
        cp_v.start()
        cp_wo = []
        for t, org in enumerate((my_pos, left, right, opp)):
            c = pltpu.make_async_copy(
                wo_hbm.at[pl.ds(org * D_LOC, D_LOC), :], wov.at[t],
                dma_sems.at[4 + t])
            c.start()
            cp_wo.append(c)

        barrier_sem = pltpu.get_barrier_semaphore()
        for nbr in (left, right):
            pl.semaphore_signal(
                barrier_sem, inc=1,
                device_id=(nbr,), device_id_type=pl.DeviceIdType.MESH,
            )
        pl.semaphore_wait(barrier_sem, 2)

        cp_x.wait()
        cp_wq.wait()
        cp_k.wait()
        cp_v.wait()
        wq_bf = wqv[:].astype(jnp.bfloat16)
        ctx = []
        for b in range(B):
            xb = xv[b].astype(jnp.bfloat16)
            q_all = jnp.dot(xb, wq_bf,
                            preferred_element_type=jnp.float32)
            q_all = q_all.astype(jnp.bfloat16)
            ctxs = []
            for j in range(H_LOC):
                q = q_all[:, j * DH:(j + 1) * DH]
                kT = kv[b, j].astype(jnp.bfloat16)
                vT = vv[b, j].astype(jnp.bfloat16)
                s = lax.dot_general(
                    q, kT, (((1,), (0,)), ((), ())),
                    preferred_element_type=jnp.float32) * 0.125
                m = jnp.max(s, axis=-1, keepdims=True)
                e = jnp.exp(s - m)
                w = (e / jnp.sum(e, axis=-1, keepdims=True)).astype(jnp.bfloat16)
                ctxs.append(lax.dot_general(
                    w, vT, (((1,), (1,)), ((), ())),
                    preferred_element_type=jnp.float32))
            cb = jnp.concatenate(ctxs, axis=1).astype(jnp.bfloat16)
            ctx.append(cb)
            comm_ref[0, b] = cb

        s0 = pltpu.make_async_remote_copy(
            src_ref=comm_ref.at[0], dst_ref=comm_ref.at[1],
            send_sem=send_sems.at[0], recv_sem=recv_sems.at[0],
            device_id=(right,), device_id_type=pl.DeviceIdType.MESH,
        )
        s1 = pltpu.make_async_remote_copy(
            src_ref=comm_ref.at[0], dst_ref=comm_ref.at[2],
            send_sem=send_sems.at[1], recv_sem=recv_sems.at[1],
            device_id=(left,), device_id_type=pl.DeviceIdType.MESH,
        )
        s0.start()
        s1.start()

        cp_wo[0].wait()
        wo_own = wov[0].astype(jnp.bfloat16)
        acc = [jnp.dot(ctx[b], wo_own, preferred_element_type=jnp.float32)
               for b in range(B)]

        s3 = pltpu.make_async_remote_copy(
            src_ref=comm_ref.at[1, 1], dst_ref=comm_ref.at[3, 1],
            send_sem=send_sems.at[3], recv_sem=recv_sems.at[3],
            device_id=(right,), device_id_type=pl.DeviceIdType.MESH,
        )
        s2 = pltpu.make_async_remote_copy(
            src_ref=comm_ref.at[2, 0], dst_ref=comm_ref.at[3, 0],
            send_sem=send_sems.at[2], recv_sem=recv_sems.at[2],
            device_id=(left,), device_id_type=pl.DeviceIdType.MESH,
        )
        s0.wait_recv()
        s3.start()
        s1.wait_recv()
        s2.start()

        cp_wo[1].wait()
        cp_wo[2].wait()
        wo_left = wov[1].astype(jnp.bfloat16)
        wo_right = wov[2].astype(jnp.bfloat16)
        for b in range(B):
            acc[b] = acc[b] + jnp.dot(
                comm_ref[1, b], wo_left, preferred_element_type=jnp.float32)
            acc[b] = acc[b] + jnp.dot(
                comm_ref[2, b], wo_right, preferred_element_type=jnp.float32)

        s2.wait_recv()
        s3.wait_recv()
        cp_wo[3].wait()
        wo_opp = wov[3].astype(jnp.bfloat16)
        for b in range(B):
            acc[b] = acc[b] + jnp.dot(
                comm_ref[3, b], wo_opp, preferred_element_type=jnp.float32)
            outv[b] = acc[b]

        cp_out = pltpu.make_async_copy(outv, out_hbm, dma_sems.at[8])
        cp_out.start()
        cp_out.wait()

        s0.wait_send()
        s1.wait_send()
        s2.wait_send()
        s3.wait_send()

    return pl.pallas_call(
        body,
        out_shape=jax.ShapeDtypeStruct((B, SQ, D_MODEL), jnp.float32),
        in_specs=[pl.BlockSpec(memory_space=pl.ANY)] * 5,
        out_specs=pl.BlockSpec(memory_space=pl.ANY),
        scratch_shapes=[
            pltpu.VMEM((B, SQ, D_MODEL), jnp.float32),
            pltpu.VMEM((D_MODEL, D_LOC), jnp.float32),
            pltpu.VMEM((B, H_LOC, DH, SQ), jnp.float32),
            pltpu.VMEM((B, H_LOC, DH, SQ), jnp.float32),
            pltpu.VMEM((4, D_LOC, D_MODEL), jnp.float32),
            pltpu.VMEM((B, SQ, D_MODEL), jnp.float32),
            pltpu.VMEM((4, B, SQ, D_LOC), jnp.bfloat16),
            pltpu.SemaphoreType.DMA((9,)),
            pltpu.SemaphoreType.DMA((4,)),
            pltpu.SemaphoreType.DMA((4,)),
        ],
        compiler_params=pltpu.CompilerParams(collective_id=0),
    )(x, Wq, K_t, V_t, Wo)


# device time: 11729 ns/iter; 3.0253x vs baseline; 1.1107x over previous
import jax
import jax.numpy as jnp
from jax import lax
from jax.experimental import pallas as pl
from jax.experimental.pallas import tpu as pltpu

N_DEV = 4
B = 2
SQ = 128
DH = 64
H_LOC = 4
D_LOC = H_LOC * DH
D_MODEL = 512


def kernel(x, Wq, K_ext, V_ext, Wo):
    K_t = jnp.transpose(K_ext, (0, 2, 3, 1))
    V_t = jnp.transpose(V_ext, (0, 2, 3, 1))

    hbm = lambda a: pltpu.with_memory_space_constraint(a, pltpu.MemorySpace.HBM)
    x, Wq, K_t, V_t, Wo = hbm(x), hbm(Wq), hbm(K_t), hbm(V_t), hbm(Wo)

    def body(x_hbm, wq_hbm, k_hbm, v_hbm, wo_hbm, out_hbm,
             xv, wqv, kv, vv, wov, outv, comm_ref,
             dma_sems, send_sems, recv_sems):
        my_pos = lax.axis_index("i")
        left = lax.rem(my_pos + N_DEV - 1, N_DEV)
        right = lax.rem(my_pos + 1, N_DEV)
        opp = lax.rem(my_pos + 2, N_DEV)
        col0 = my_pos * D_LOC

        cp_x = pltpu.make_async_copy(x_hbm, xv, dma_sems.at[0])
        cp_x.start()
        cp_wq = pltpu.make_async_copy(
            wq_hbm.at[:, pl.ds(col0, D_LOC)], wqv, dma_sems.at[1])
        cp_wq.start()
        cp_k = pltpu.make_async_copy(k_hbm, kv, dma_sems.at[2])
        cp_k.start()
        cp_v = pltpu.make_async_copy(v_hbm, vv, dma_sems.at[3])
        cp_v.start()
        cp_wo = []
        for t, org in enumerate((my_pos, left, right, opp)):
            c = pltpu.make_async_copy(
                wo_hbm.at[pl.ds(org * D_LOC, D_LOC), :], wov.at[t],
                dma_sems.at[4 + t])
            c.start()
            cp_wo.append(c)

        barrier_sem = pltpu.get_barrier_semaphore()
        for nbr in (left, right):
            pl.semaphore_signal(
                barrier_sem, inc=1,
                device_id=(nbr,), device_id_type=pl.DeviceIdType.MESH,
            )
        pl.semaphore_wait(barrier_sem, 2)

        cp_x.wait()
        cp_wq.wait()
        cp_k.wait()
        cp_v.wait()
        wq_bf = wqv[:].astype(jnp.bfloat16)

        def attn_batch(b):
            xb = xv[b].astype(jnp.bfloat16)
            q_all = jnp.dot(xb, wq_bf,
                            preferred_element_type=jnp.float32)
            q_all = q_all.astype(jnp.bfloat16)
            ctxs = []
            for j in range(H_LOC):
                q = q_all[:, j * DH:(j + 1) * DH]
                kT = kv[b, j].astype(jnp.bfloat16)
                vT = vv[b, j].astype(jnp.bfloat16)
                s = lax.dot_general(
                    q, kT, (((1,), (0,)), ((), ())),
                    preferred_element_type=jnp.float32) * 0.125
                e = jnp.exp(s)
                r = 1.0 / jnp.sum(e, axis=-1, keepdims=True)
                cj = lax.dot_general(
                    e.astype(jnp.bfloat16), vT, (((1,), (1,)), ((), ())),
                    preferred_element_type=jnp.float32)
                ctxs.append(cj * r)
            return jnp.concatenate(ctxs, axis=1).astype(jnp.bfloat16)

        def p1_sends(b):
            sr = pltpu.make_async_remote_copy(
                src_ref=comm_ref.at[0, b], dst_ref=comm_ref.at[1, b],
                send_sem=send_sems.at[2 * b], recv_sem=recv_sems.at[2 * b],
                device_id=(right,), device_id_type=pl.DeviceIdType.MESH,
            )
            sl = pltpu.make_async_remote_copy(
                src_ref=comm_ref.at[0, b], dst_ref=comm_ref.at[2, b],
                send_sem=send_sems.at[2 * b + 1],
                recv_sem=recv_sems.at[2 * b + 1],
                device_id=(left,), device_id_type=pl.DeviceIdType.MESH,
            )
            sr.start()
            sl.start()
            return sr, sl

        ctx0 = attn_batch(0)
        comm_ref[0, 0] = ctx0
        sr0, sl0 = p1_sends(0)
        ctx1 = attn_batch(1)
        comm_ref[0, 1] = ctx1
        sr1, sl1 = p1_sends(1)

        cp_wo[0].wait()
        wo_own = wov[0].astype(jnp.bfloat16)
        acc = [jnp.dot(c, wo_own, preferred_element_type=jnp.float32)
               for c in (ctx0, ctx1)]

        fwd_l = pltpu.make_async_remote_copy(
            src_ref=comm_ref.at[2, 0], dst_ref=comm_ref.at[3, 0],
            send_sem=send_sems.at[4], recv_sem=recv_sems.at[4],
            device_id=(left,), device_id_type=pl.DeviceIdType.MESH,
        )
        fwd_r = pltpu.make_async_remote_copy(
            src_ref=comm_ref.at[1, 1], dst_ref=comm_ref.at[3, 1],
            send_sem=send_sems.at[5], recv_sem=recv_sems.at[5],
            device_id=(right,), device_id_type=pl.DeviceIdType.MESH,
        )
        sl0.wait_recv()
        fwd_l.start()
        sr1.wait_recv()
        fwd_r.start()
        sr0.wait_recv()
        sl1.wait_recv()

        cp_wo[1].wait()
        cp_wo[2].wait()
        wo_left = wov[1].astype(jnp.bfloat16)
        wo_right = wov[2].astype(jnp.bfloat16)
        for b in range(B):
            acc[b] = acc[b] + jnp.dot(
                comm_ref[1, b], wo_left, preferred_element_type=jnp.float32)
            acc[b] = acc[b] + jnp.dot(
                comm_ref[2, b], wo_right, preferred_element_type=jnp.float32)

        fwd_l.wait_recv()
        fwd_r.wait_recv()
        cp_wo[3].wait()
        wo_opp = wov[3].astype(jnp.bfloat16)
        for b in range(B):
            acc[b] = acc[b] + jnp.dot(
                comm_ref[3, b], wo_opp, preferred_element_type=jnp.float32)
            outv[b] = acc[b]

        cp_out = pltpu.make_async_copy(outv, out_hbm, dma_sems.at[8])
        cp_out.start()
        cp_out.wait()

        for s in (sr0, sl0, sr1, sl1, fwd_l, fwd_r):
            s.wait_send()

    return pl.pallas_call(
        body,
        out_shape=jax.ShapeDtypeStruct((B, SQ, D_MODEL), jnp.float32),
        in_specs=[pl.BlockSpec(memory_space=pl.ANY)] * 5,
        out_specs=pl.BlockSpec(memory_space=pltpu.MemorySpace.HBM),
        scratch_shapes=[
            pltpu.VMEM((B, SQ, D_MODEL), jnp.float32),
            pltpu.VMEM((D_MODEL, D_LOC), jnp.float32),
            pltpu.VMEM((B, H_LOC, DH, SQ), jnp.float32),
            pltpu.VMEM((B, H_LOC, DH, SQ), jnp.float32),
            pltpu.VMEM((4, D_LOC, D_MODEL), jnp.float32),
            pltpu.VMEM((B, SQ, D_MODEL), jnp.float32),
            pltpu.VMEM((4, B, SQ, D_LOC), jnp.bfloat16),
            pltpu.SemaphoreType.DMA((9,)),
            pltpu.SemaphoreType.DMA((6,)),
            pltpu.SemaphoreType.DMA((6,)),
        ],
        compiler_params=pltpu.CompilerParams(collective_id=0),
    )(x, Wq, K_t, V_t, Wo)


# device time: 11721 ns/iter; 3.0274x vs baseline; 1.0007x over previous
import jax
import jax.numpy as jnp
from jax import lax
from jax.experimental import pallas as pl
from jax.experimental.pallas import tpu as pltpu

N_DEV = 4
B = 2
SQ = 128
DH = 64
H_LOC = 4
D_LOC = H_LOC * DH
D_MODEL = 512


def kernel(x, Wq, K_ext, V_ext, Wo):
    K_t = jnp.transpose(K_ext, (0, 2, 3, 1))
    V_t = jnp.transpose(V_ext, (0, 2, 3, 1))

    hbm = lambda a: pltpu.with_memory_space_constraint(a, pltpu.MemorySpace.HBM)
    x, Wq, K_t, V_t, Wo = hbm(x), hbm(Wq), hbm(K_t), hbm(V_t), hbm(Wo)

    def body(x_hbm, wq_hbm, k_hbm, v_hbm, wo_hbm, out_ref,
             xv, wqv, kv, vv, wov, comm_ref,
             dma_sems, send_sems, recv_sems):
        my_pos = lax.axis_index("i")
        left = lax.rem(my_pos + N_DEV - 1, N_DEV)
        right = lax.rem(my_pos + 1, N_DEV)
        opp = lax.rem(my_pos + 2, N_DEV)
        col0 = my_pos * D_LOC

        barrier_sem = pltpu.get_barrier_semaphore()
        for nbr in (left, right):
            pl.semaphore_signal(
                barrier_sem, inc=1,
                device_id=(nbr,), device_id_type=pl.DeviceIdType.MESH,
            )

        cp_x = pltpu.make_async_copy(x_hbm, xv, dma_sems.at[0])
        cp_x.start()
        cp_wq = pltpu.make_async_copy(
            wq_hbm.at[:, pl.ds(col0, D_LOC)], wqv, dma_sems.at[1])
        cp_wq.start()
        cp_k = pltpu.make_async_copy(k_hbm, kv, dma_sems.at[2])
        cp_k.start()
        cp_v = pltpu.make_async_copy(v_hbm, vv, dma_sems.at[3])
        cp_v.start()
        cp_wo = []
        for t, org in enumerate((my_pos, left, right, opp)):
            c = pltpu.make_async_copy(
                wo_hbm.at[pl.ds(org * D_LOC, D_LOC), :], wov.at[t],
                dma_sems.at[4 + t])
            c.start()
            cp_wo.append(c)

        cp_x.wait()
        cp_wq.wait()
        cp_k.wait()
        cp_v.wait()
        wq_bf = wqv[:].astype(jnp.bfloat16)

        def attn_batch(b):
            xb = xv[b].astype(jnp.bfloat16)
            q_all = jnp.dot(xb, wq_bf,
                            preferred_element_type=jnp.float32)
            q_all = q_all.astype(jnp.bfloat16)
            ctxs = []
            for j in range(H_LOC):
                q = q_all[:, j * DH:(j + 1) * DH]
                kT = kv[b, j].astype(jnp.bfloat16)
                vT = vv[b, j].astype(jnp.bfloat16)
                s = lax.dot_general(
                    q, kT, (((1,), (0,)), ((), ())),
                    preferred_element_type=jnp.float32) * 0.125
                e = jnp.exp(s)
                r = 1.0 / jnp.sum(e, axis=-1, keepdims=True)
                cj = lax.dot_general(
                    e.astype(jnp.bfloat16), vT, (((1,), (1,)), ((), ())),
                    preferred_element_type=jnp.float32)
                ctxs.append(cj * r)
            return jnp.concatenate(ctxs, axis=1).astype(jnp.bfloat16)

        def p1_sends(b):
            sr = pltpu.make_async_remote_copy(
                src_ref=comm_ref.at[0, b], dst_ref=comm_ref.at[1, b],
                send_sem=send_sems.at[2 * b], recv_sem=recv_sems.at[2 * b],
                device_id=(right,), device_id_type=pl.DeviceIdType.MESH,
            )
            sl = pltpu.make_async_remote_copy(
                src_ref=comm_ref.at[0, b], dst_ref=comm_ref.at[2, b],
                send_sem=send_sems.at[2 * b + 1],
                recv_sem=recv_sems.at[2 * b + 1],
                device_id=(left,), device_id_type=pl.DeviceIdType.MESH,
            )
            sr.start()
            sl.start()
            return sr, sl

        ctx0 = attn_batch(0)
        comm_ref[0, 0] = ctx0
        pl.semaphore_wait(barrier_sem, 2)
        sr0, sl0 = p1_sends(0)
        ctx1 = attn_batch(1)
        comm_ref[0, 1] = ctx1
        sr1, sl1 = p1_sends(1)

        cp_wo[0].wait()
        wo_own = wov[0].astype(jnp.bfloat16)
        acc = [jnp.dot(c, wo_own, preferred_element_type=jnp.float32)
               for c in (ctx0, ctx1)]

        fwd_l = pltpu.make_async_remote_copy(
            src_ref=comm_ref.at[2, 0], dst_ref=comm_ref.at[3, 0],
            send_sem=send_sems.at[4], recv_sem=recv_sems.at[4],
            device_id=(left,), device_id_type=pl.DeviceIdType.MESH,
        )
        fwd_r = pltpu.make_async_remote_copy(
            src_ref=comm_ref.at[1, 1], dst_ref=comm_ref.at[3, 1],
            send_sem=send_sems.at[5], recv_sem=recv_sems.at[5],
            device_id=(right,), device_id_type=pl.DeviceIdType.MESH,
        )
        sl0.wait_recv()
        fwd_l.start()
        sr1.wait_recv()
        fwd_r.start()
        sr0.wait_recv()
        sl1.wait_recv()

        cp_wo[1].wait()
        cp_wo[2].wait()
        wo_left = wov[1].astype(jnp.bfloat16)
        wo_right = wov[2].astype(jnp.bfloat16)
        for b in range(B):
            acc[b] = acc[b] + jnp.dot(
                comm_ref[1, b], wo_left, preferred_element_type=jnp.float32)
            acc[b] = acc[b] + jnp.dot(
                comm_ref[2, b], wo_right, preferred_element_type=jnp.float32)

        fwd_l.wait_recv()
        fwd_r.wait_recv()
        cp_wo[3].wait()
        wo_opp = wov[3].astype(jnp.bfloat16)
        for b in range(B):
            acc[b] = acc[b] + jnp.dot(
                comm_ref[3, b], wo_opp, preferred_element_type=jnp.float32)
            out_ref[b] = acc[b]

        for s in (sr0, sl0, sr1, sl1, fwd_l, fwd_r):
            s.wait_send()

    return pl.pallas_call(
        body,
        out_shape=jax.ShapeDtypeStruct((B, SQ, D_MODEL), jnp.float32),
        in_specs=[pl.BlockSpec(memory_space=pl.ANY)] * 5,
        out_specs=pl.BlockSpec(memory_space=pltpu.VMEM),
        scratch_shapes=[
            pltpu.VMEM((B, SQ, D_MODEL), jnp.float32),
            pltpu.VMEM((D_MODEL, D_LOC), jnp.float32),
            pltpu.VMEM((B, H_LOC, DH, SQ), jnp.float32),
            pltpu.VMEM((B, H_LOC, DH, SQ), jnp.float32),
            pltpu.VMEM((4, D_LOC, D_MODEL), jnp.float32),
            pltpu.VMEM((4, B, SQ, D_LOC), jnp.bfloat16),
            pltpu.SemaphoreType.DMA((8,)),
            pltpu.SemaphoreType.DMA((6,)),
            pltpu.SemaphoreType.DMA((6,)),
        ],
        compiler_params=pltpu.CompilerParams(collective_id=0),
    )(x, Wq, K_t, V_t, Wo)


# device time: 10700 ns/iter; 3.3163x vs baseline; 1.0954x over previous
import jax
import jax.numpy as jnp
from jax import lax
from jax.experimental import pallas as pl
from jax.experimental.pallas import tpu as pltpu

N_DEV = 4
B = 2
SQ = 128
DH = 64
H_LOC = 4
D_LOC = H_LOC * DH
D_MODEL = 512


def kernel(x, Wq, K_ext, V_ext, Wo):
    K_t = jnp.transpose(K_ext, (0, 2, 3, 1))
    V_t = jnp.transpose(V_ext, (0, 2, 3, 1))

    hbm = lambda a: pltpu.with_memory_space_constraint(a, pltpu.MemorySpace.HBM)
    x, Wq, K_t, V_t, Wo = hbm(x), hbm(Wq), hbm(K_t), hbm(V_t), hbm(Wo)

    def body(x_hbm, wq_hbm, k_hbm, v_hbm, wo_hbm, out_ref,
             xv, wqv, kv, vv, wov, comm_ref,
             dma_sems, send_sems, recv_sems):
        my_pos = lax.axis_index("i")
        left = lax.rem(my_pos + N_DEV - 1, N_DEV)
        right = lax.rem(my_pos + 1, N_DEV)
        opp = lax.rem(my_pos + 2, N_DEV)
        col0 = my_pos * D_LOC

        barrier_sem = pltpu.get_barrier_semaphore()
        for nbr in (left, right, opp):
            pl.semaphore_signal(
                barrier_sem, inc=1,
                device_id=(nbr,), device_id_type=pl.DeviceIdType.MESH,
            )

        cp_x = pltpu.make_async_copy(x_hbm, xv, dma_sems.at[0])
        cp_x.start()
        cp_wq = pltpu.make_async_copy(
            wq_hbm.at[:, pl.ds(col0, D_LOC)], wqv, dma_sems.at[1])
        cp_wq.start()
        cp_k = pltpu.make_async_copy(k_hbm, kv, dma_sems.at[2])
        cp_k.start()
        cp_v = pltpu.make_async_copy(v_hbm, vv, dma_sems.at[3])
        cp_v.start()
        cp_wo = []
        for t, org in enumerate((my_pos, left, right, opp)):
            c = pltpu.make_async_copy(
                wo_hbm.at[pl.ds(org * D_LOC, D_LOC), :], wov.at[t],
                dma_sems.at[4 + t])
            c.start()
            cp_wo.append(c)

        cp_x.wait()
        cp_wq.wait()
        cp_k.wait()
        cp_v.wait()
        wq_bf = wqv[:].astype(jnp.bfloat16)

        def attn_batch(b):
            xb = xv[b].astype(jnp.bfloat16)
            q_all = jnp.dot(xb, wq_bf,
                            preferred_element_type=jnp.float32)
            q_all = q_all.astype(jnp.bfloat16)
            ctxs = []
            for j in range(H_LOC):
                q = q_all[:, j * DH:(j + 1) * DH]
                kT = kv[b, j].astype(jnp.bfloat16)
                vT = vv[b, j].astype(jnp.bfloat16)
                s = lax.dot_general(
                    q, kT, (((1,), (0,)), ((), ())),
                    preferred_element_type=jnp.float32) * 0.125
                e = jnp.exp(s)
                r = 1.0 / jnp.sum(e, axis=-1, keepdims=True)
                cj = lax.dot_general(
                    e.astype(jnp.bfloat16), vT, (((1,), (1,)), ((), ())),
                    preferred_element_type=jnp.float32)
                ctxs.append(cj * r)
            return jnp.concatenate(ctxs, axis=1).astype(jnp.bfloat16)

        def pushes(b):
            out = []
            for i, (tgt, slot) in enumerate(((opp, 3), (right, 1), (left, 2))):
                r = pltpu.make_async_remote_copy(
                    src_ref=comm_ref.at[0, b], dst_ref=comm_ref.at[slot, b],
                    send_sem=send_sems.at[3 * b + i],
                    recv_sem=recv_sems.at[3 * b + i],
                    device_id=(tgt,), device_id_type=pl.DeviceIdType.MESH,
                )
                r.start()
                out.append(r)
            return out

        ctx0 = attn_batch(0)
        comm_ref[0, 0] = ctx0
        pl.semaphore_wait(barrier_sem, 3)
        p0 = pushes(0)
        ctx1 = attn_batch(1)
        comm_ref[0, 1] = ctx1
        p1 = pushes(1)

        cp_wo[0].wait()
        wo_own = wov[0].astype(jnp.bfloat16)
        acc = [jnp.dot(c, wo_own, preferred_element_type=jnp.float32)
               for c in (ctx0, ctx1)]

        cp_wo[1].wait()
        cp_wo[2].wait()
        wo_left = wov[1].astype(jnp.bfloat16)
        wo_right = wov[2].astype(jnp.bfloat16)
        p0[2].wait_recv()
        p0[1].wait_recv()
        acc[0] = acc[0] + jnp.dot(
            comm_ref[1, 0], wo_left, preferred_element_type=jnp.float32)
        acc[0] = acc[0] + jnp.dot(
            comm_ref[2, 0], wo_right, preferred_element_type=jnp.float32)
        p1[2].wait_recv()
        p1[1].wait_recv()
        acc[1] = acc[1] + jnp.dot(
            comm_ref[1, 1], wo_left, preferred_element_type=jnp.float32)
        acc[1] = acc[1] + jnp.dot(
            comm_ref[2, 1], wo_right, preferred_element_type=jnp.float32)

        cp_wo[3].wait()
        wo_opp = wov[3].astype(jnp.bfloat16)
        p0[0].wait_recv()
        acc[0] = acc[0] + jnp.dot(
            comm_ref[3, 0], wo_opp, preferred_element_type=jnp.float32)
        out_ref[0] = acc[0]
        p1[0].wait_recv()
        acc[1] = acc[1] + jnp.dot(
            comm_ref[3, 1], wo_opp, preferred_element_type=jnp.float32)
        out_ref[1] = acc[1]

        for r in p0 + p1:
            r.wait_send()

    return pl.pallas_call(
        body,
        out_shape=jax.ShapeDtypeStruct((B, SQ, D_MODEL), jnp.float32),
        in_specs=[pl.BlockSpec(memory_space=pl.ANY)] * 5,
        out_specs=pl.BlockSpec(memory_space=pltpu.VMEM),
        scratch_shapes=[
            pltpu.VMEM((B, SQ, D_MODEL), jnp.float32),
            pltpu.VMEM((D_MODEL, D_LOC), jnp.float32),
            pltpu.VMEM((B, H_LOC, DH, SQ), jnp.float32),
            pltpu.VMEM((B, H_LOC, DH, SQ), jnp.float32),
            pltpu.VMEM((4, D_LOC, D_MODEL), jnp.float32),
            pltpu.VMEM((4, B, SQ, D_LOC), jnp.bfloat16),
            pltpu.SemaphoreType.DMA((8,)),
            pltpu.SemaphoreType.DMA((6,)),
            pltpu.SemaphoreType.DMA((6,)),
        ],
        compiler_params=pltpu.CompilerParams(collective_id=0),
    )(x, Wq, K_t, V_t, Wo)


# device time: 10250 ns/iter; 3.4619x vs baseline; 1.0439x over previous
import jax
import jax.numpy as jnp
from jax import lax
from jax.experimental import pallas as pl
from jax.experimental.pallas import tpu as pltpu

N_DEV = 4
B = 2
SQ = 128
DH = 64
H_LOC = 4
D_LOC = H_LOC * DH
D_MODEL = 512


def kernel(x, Wq, K_ext, V_ext, Wo):
    K_t = jnp.transpose(K_ext, (0, 2, 3, 1))
    V_t = jnp.transpose(V_ext, (0, 2, 3, 1))

    hbm = lambda a: pltpu.with_memory_space_constraint(a, pltpu.MemorySpace.HBM)
    x, Wq, K_t, V_t, Wo = hbm(x), hbm(Wq), hbm(K_t), hbm(V_t), hbm(Wo)

    def body(x_hbm, wq_hbm, k_hbm, v_hbm, wo_hbm, out_ref,
             xv, wqv, kv, vv, wov, comm_ref,
             dma_sems, send_sems, recv_sems):
        my_pos = lax.axis_index("i")
        left = lax.rem(my_pos + N_DEV - 1, N_DEV)
        right = lax.rem(my_pos + 1, N_DEV)
        opp = lax.rem(my_pos + 2, N_DEV)
        col0 = my_pos * D_LOC

        barrier_sem = pltpu.get_barrier_semaphore()
        for nbr in (left, right, opp):
            pl.semaphore_signal(
                barrier_sem, inc=1,
                device_id=(nbr,), device_id_type=pl.DeviceIdType.MESH,
            )

        cp_x = pltpu.make_async_copy(x_hbm, xv, dma_sems.at[0])
        cp_x.start()
        cp_wq = pltpu.make_async_copy(
            wq_hbm.at[:, pl.ds(col0, D_LOC)], wqv, dma_sems.at[1])
        cp_wq.start()
        cp_k = pltpu.make_async_copy(k_hbm, kv, dma_sems.at[2])
        cp_k.start()
        cp_v = pltpu.make_async_copy(v_hbm, vv, dma_sems.at[3])
        cp_v.start()
        cp_wo = []
        for t, org in enumerate((my_pos, left, right, opp)):
            c = pltpu.make_async_copy(
                wo_hbm.at[pl.ds(org * D_LOC, D_LOC), :], wov.at[t],
                dma_sems.at[4 + t])
            c.start()
            cp_wo.append(c)

        cp_x.wait()
        cp_wq.wait()
        wq_bf = wqv[:].astype(jnp.bfloat16)
        q_alls = []
        for b in range(B):
            xb = xv[b].astype(jnp.bfloat16)
            qa = jnp.dot(xb, wq_bf,
                         preferred_element_type=jnp.float32)
            q_alls.append(qa.astype(jnp.bfloat16))
        cp_k.wait()
        cp_v.wait()

        def attn_batch(b):
            q_all = q_alls[b]
            ctxs = []
            for j in range(H_LOC):
                q = q_all[:, j * DH:(j + 1) * DH]
                kT = kv[b, j].astype(jnp.bfloat16)
                vT = vv[b, j].astype(jnp.bfloat16)
                s = lax.dot_general(
                    q, kT, (((1,), (0,)), ((), ())),
                    preferred_element_type=jnp.float32) * 0.125
                e = jnp.exp(s)
                r = 1.0 / jnp.sum(e, axis=-1, keepdims=True)
                cj = lax.dot_general(
                    e.astype(jnp.bfloat16), vT, (((1,), (1,)), ((), ())),
                    preferred_element_type=jnp.float32)
                ctxs.append(cj * r)
            return jnp.concatenate(ctxs, axis=1).astype(jnp.bfloat16)

        def pushes(b):
            out = []
            for i, (tgt, slot) in enumerate(((opp, 3), (right, 1), (left, 2))):
                r = pltpu.make_async_remote_copy(
                    src_ref=comm_ref.at[0, b], dst_ref=comm_ref.at[slot, b],
                    send_sem=send_sems.at[3 * b + i],
                    recv_sem=recv_sems.at[3 * b + i],
                    device_id=(tgt,), device_id_type=pl.DeviceIdType.MESH,
                )
                r.start()
                out.append(r)
            return out

        ctx0 = attn_batch(0)
        comm_ref[0, 0] = ctx0
        pl.semaphore_wait(barrier_sem, 3)
        p0 = pushes(0)
        ctx1 = attn_batch(1)
        comm_ref[0, 1] = ctx1
        p1 = pushes(1)

        cp_wo[0].wait()
        wo_own = wov[0].astype(jnp.bfloat16)
        acc = [jnp.dot(c, wo_own, preferred_element_type=jnp.float32)
               for c in (ctx0, ctx1)]

        cp_wo[1].wait()
        cp_wo[2].wait()
        wo_left = wov[1].astype(jnp.bfloat16)
        wo_right = wov[2].astype(jnp.bfloat16)
        p0[2].wait_recv()
        p0[1].wait_recv()
        acc[0] = acc[0] + jnp.dot(
            comm_ref[1, 0], wo_left, preferred_element_type=jnp.float32)
        acc[0] = acc[0] + jnp.dot(
            comm_ref[2, 0], wo_right, preferred_element_type=jnp.float32)
        p1[2].wait_recv()
        p1[1].wait_recv()
        acc[1] = acc[1] + jnp.dot(
            comm_ref[1, 1], wo_left, preferred_element_type=jnp.float32)
        acc[1] = acc[1] + jnp.dot(
            comm_ref[2, 1], wo_right, preferred_element_type=jnp.float32)

        cp_wo[3].wait()
        wo_opp = wov[3].astype(jnp.bfloat16)
        p0[0].wait_recv()
        acc[0] = acc[0] + jnp.dot(
            comm_ref[3, 0], wo_opp, preferred_element_type=jnp.float32)
        out_ref[0] = acc[0]
        p1[0].wait_recv()
        acc[1] = acc[1] + jnp.dot(
            comm_ref[3, 1], wo_opp, preferred_element_type=jnp.float32)
        out_ref[1] = acc[1]

        for r in p0 + p1:
            r.wait_send()

    return pl.pallas_call(
        body,
        out_shape=jax.ShapeDtypeStruct((B, SQ, D_MODEL), jnp.float32),
        in_specs=[pl.BlockSpec(memory_space=pl.ANY)] * 5,
        out_specs=pl.BlockSpec(memory_space=pltpu.VMEM),
        scratch_shapes=[
            pltpu.VMEM((B, SQ, D_MODEL), jnp.float32),
            pltpu.VMEM((D_MODEL, D_LOC), jnp.float32),
            pltpu.VMEM((B, H_LOC, DH, SQ), jnp.float32),
            pltpu.VMEM((B, H_LOC, DH, SQ), jnp.float32),
            pltpu.VMEM((4, D_LOC, D_MODEL), jnp.float32),
            pltpu.VMEM((4, B, SQ, D_LOC), jnp.bfloat16),
            pltpu.SemaphoreType.DMA((8,)),
            pltpu.SemaphoreType.DMA((6,)),
            pltpu.SemaphoreType.DMA((6,)),
        ],
        compiler_params=pltpu.CompilerParams(collective_id=0),
    )(x, Wq, K_t, V_t, Wo)
